# Initial kernel scaffold; baseline (speedup 1.0000x reference)
import jax
import jax.numpy as jnp
from jax.experimental import pallas as pl


def kernel(x, edge_index, edge_weight, W1, b1, W2, b2):
    raise NotImplementedError("write your pallas kernel here")


# speedup vs baseline: 2.3286x; 2.3286x over previous
import jax
import jax.numpy as jnp
from jax import lax
from jax.experimental import pallas as pl
from jax.experimental.pallas import tpu as pltpu
from jax.experimental.pallas import tpu_sc as plsc

NC = 2
NS = 16
L = 16
NW = NC * NS


def _mesh():
    return plsc.VectorSubcoreMesh(
        core_axis_name="c", subcore_axis_name="s", num_cores=NC,
        num_subcores=NS)


def _deg_kernel(n_pad, rows128):
    nrows_per_w = rows128 // NW
    npt = n_pad // NS

    def body(col_h, ew_h, deg_h, deg_s, colb, ewb, zb):
        cid = lax.axis_index("c")
        sid = lax.axis_index("s")
        wid = cid * NS + sid

        def fill_z(i, _):
            zb[pl.ds(i * L, L)] = jnp.zeros((L,), jnp.float32)
            return 0
        lax.fori_loop(0, npt // L, fill_z, 0)
        pltpu.sync_copy(zb, deg_s.at[pl.ds(sid * npt, npt)])
        plsc.subcore_barrier()

        def deg_chunk(b, _):
            base = wid * nrows_per_w + b * 16
            pltpu.sync_copy(col_h.at[pl.ds(base, 16)], colb)
            pltpu.sync_copy(ew_h.at[pl.ds(base, 16)], ewb)
            def deg_row(j, _):
                pltpu.sync_copy(ewb.at[j], deg_s.at[colb.at[j]], add=True)
                return 0
            lax.fori_loop(0, 16, deg_row, 0)
            return 0
        lax.fori_loop(0, nrows_per_w // 16, deg_chunk, 0)
        plsc.subcore_barrier()

        pltpu.sync_copy(deg_s.at[pl.ds(sid * npt, npt)],
                        deg_h.at[pl.ds(cid * n_pad + sid * npt, npt)])

    return pl.kernel(
        body,
        out_type=jax.ShapeDtypeStruct((NC * n_pad,), jnp.float32),
        mesh=_mesh(),
        scratch_types=[
            pltpu.VMEM_SHARED((n_pad,), jnp.float32),
            pltpu.VMEM((16, 128), jnp.int32),
            pltpu.VMEM((16, 128), jnp.float32),
            pltpu.VMEM((n_pad // NS,), jnp.float32),
        ],
        compiler_params=pltpu.CompilerParams(needs_layout_passes=False),
        name="gcn_deg_sc",
    )


def _norm_kernel(n_pad, rows128):
    nrows_per_w = rows128 // NW

    def body(dinv_h, row_h, col_h, ew_h, norm_h, rowb, colb, ewb, normb,
             dinv_all):
        cid = lax.axis_index("c")
        sid = lax.axis_index("s")
        wid = cid * NS + sid

        pltpu.sync_copy(dinv_h, dinv_all)

        def norm_chunk(b, _):
            base = wid * nrows_per_w + b * 16
            pltpu.sync_copy(row_h.at[pl.ds(base, 16)], rowb)
            pltpu.sync_copy(col_h.at[pl.ds(base, 16)], colb)
            pltpu.sync_copy(ew_h.at[pl.ds(base, 16)], ewb)
            def norm_row(j, _):
                for g in range(8):
                    rr = rowb[j, pl.ds(g * L, L)]
                    cc = colb[j, pl.ds(g * L, L)]
                    ev = ewb[j, pl.ds(g * L, L)]
                    dr = plsc.load_gather(dinv_all, [rr])
                    dc = plsc.load_gather(dinv_all, [cc])
                    normb[j, pl.ds(g * L, L)] = dr * ev * dc
                return 0
            lax.fori_loop(0, 16, norm_row, 0)
            pltpu.sync_copy(normb, norm_h.at[pl.ds(base, 16)])
            return 0
        lax.fori_loop(0, nrows_per_w // 16, norm_chunk, 0)

    return pl.kernel(
        body,
        out_type=jax.ShapeDtypeStruct((rows128, 128), jnp.float32),
        mesh=_mesh(),
        scratch_types=[
            pltpu.VMEM((16, 128), jnp.int32),
            pltpu.VMEM((16, 128), jnp.int32),
            pltpu.VMEM((16, 128), jnp.float32),
            pltpu.VMEM((16, 128), jnp.float32),
            pltpu.VMEM((n_pad,), jnp.float32),
        ],
        compiler_params=pltpu.CompilerParams(needs_layout_passes=False),
        name="gcn_norm_sc",
    )


def _aggregate_kernel(n_pad, rows128):
    nrows_per_w = rows128 // NW
    npt = n_pad // NS

    def body(h_hbm, row_h, col_h, norm_h, out_h, acc_s, rowb, colb, normb,
             rows_v, zbuf, sem):
        cid = lax.axis_index("c")
        sid = lax.axis_index("s")
        wid = cid * NS + sid
        iotas = [lax.iota(jnp.int32, L) + g * L for g in range(8)]

        def zfill(r, _):
            for g in range(8):
                zbuf[r, pl.ds(g * L, L)] = jnp.zeros((L,), jnp.float32)
            return 0
        lax.fori_loop(0, 64, zfill, 0)
        def zcopy(k, _):
            pltpu.sync_copy(zbuf, acc_s.at[pl.ds(sid * npt + k * 64, 64)])
            return 0
        lax.fori_loop(0, npt // 64, zcopy, 0)
        plsc.subcore_barrier()

        def big_chunk(b, _):
            base = wid * nrows_per_w + b * 16
            pltpu.sync_copy(row_h.at[pl.ds(base, 16)], rowb)
            pltpu.sync_copy(col_h.at[pl.ds(base, 16)], colb)
            pltpu.sync_copy(norm_h.at[pl.ds(base, 16)], normb)
            def edge_row(j, _):
                pltpu.async_copy(h_hbm.at[rowb.at[j]], rows_v, sem).wait()
                nv = [normb[j, pl.ds(g * L, L)] for g in range(8)]
                def scale_f(f, _):
                    fv = jnp.full((L,), f, jnp.int32)
                    for g in range(8):
                        vals = plsc.load_gather(rows_v, [iotas[g], fv])
                        plsc.store_scatter(rows_v, [iotas[g], fv],
                                           vals * nv[g])
                    return 0
                lax.fori_loop(0, 128, scale_f, 0)
                pltpu.sync_copy(rows_v, acc_s.at[colb.at[j]], add=True)
                return 0
            lax.fori_loop(0, 16, edge_row, 0)
            return 0
        lax.fori_loop(0, nrows_per_w // 16, big_chunk, 0)
        plsc.subcore_barrier()

        pltpu.sync_copy(acc_s.at[pl.ds(sid * npt, npt)],
                        out_h.at[pl.ds(cid * n_pad + sid * npt, npt)])

    return pl.kernel(
        body,
        out_type=jax.ShapeDtypeStruct((NC * n_pad, 128), jnp.float32),
        mesh=_mesh(),
        scratch_types=[
            pltpu.VMEM_SHARED((n_pad, 128), jnp.float32),
            pltpu.VMEM((16, 128), jnp.int32),
            pltpu.VMEM((16, 128), jnp.int32),
            pltpu.VMEM((16, 128), jnp.float32),
            pltpu.VMEM((128, 128), jnp.float32),
            pltpu.VMEM((64, 128), jnp.float32),
            pltpu.SemaphoreType.DMA,
        ],
        compiler_params=pltpu.CompilerParams(needs_layout_passes=False),
        name="gcn_aggregate_sc",
    )


def _mm_deg(x, w, deg0, deg1, bm):
    n = x.shape[0]
    def k(xb, wb, d0b, d1b, hb, dib, d2b):
        hb[...] = jnp.dot(xb[...], wb[...],
                          preferred_element_type=jnp.float32)
        dinv = lax.rsqrt(1.0 + d0b[...] + d1b[...])
        dib[...] = dinv
        d2b[...] = dinv * dinv
    return pl.pallas_call(
        k,
        grid=(n // bm,),
        in_specs=[pl.BlockSpec((bm, 128), lambda i: (i, 0)),
                  pl.BlockSpec((128, 128), lambda i: (0, 0)),
                  pl.BlockSpec((bm, 1), lambda i: (i, 0)),
                  pl.BlockSpec((bm, 1), lambda i: (i, 0))],
        out_specs=[pl.BlockSpec((bm, 128), lambda i: (i, 0)),
                   pl.BlockSpec((bm, 1), lambda i: (i, 0)),
                   pl.BlockSpec((bm, 1), lambda i: (i, 0))],
        out_shape=[jax.ShapeDtypeStruct((n, 128), jnp.float32),
                   jax.ShapeDtypeStruct((n, 1), jnp.float32),
                   jax.ShapeDtypeStruct((n, 1), jnp.float32)],
    )(x, w, deg0, deg1)


def _fused_relu_mm(p0, p1, h, d2, b, w, bm):
    n = h.shape[0]
    def k(p0b, p1b, hb, db, bb, wb, ob):
        g = jnp.maximum(
            p0b[...] + p1b[...] + db[...] * hb[...] + bb[...], 0.0)
        ob[...] = jnp.dot(g, wb[...], preferred_element_type=jnp.float32)
    return pl.pallas_call(
        k,
        grid=(n // bm,),
        in_specs=[pl.BlockSpec((bm, 128), lambda i: (i, 0)),
                  pl.BlockSpec((bm, 128), lambda i: (i, 0)),
                  pl.BlockSpec((bm, 128), lambda i: (i, 0)),
                  pl.BlockSpec((bm, 1), lambda i: (i, 0)),
                  pl.BlockSpec((1, 128), lambda i: (0, 0)),
                  pl.BlockSpec((128, 128), lambda i: (0, 0))],
        out_specs=pl.BlockSpec((bm, 128), lambda i: (i, 0)),
        out_shape=jax.ShapeDtypeStruct((n, 128), jnp.float32),
    )(p0, p1, h, d2, b, w)


def _epilogue(q0, q1, h, d2, b, bm):
    n = h.shape[0]
    def k(q0b, q1b, hb, db, bb, ob):
        ob[...] = q0b[...] + q1b[...] + db[...] * hb[...] + bb[...]
    return pl.pallas_call(
        k,
        grid=(n // bm,),
        in_specs=[pl.BlockSpec((bm, 128), lambda i: (i, 0)),
                  pl.BlockSpec((bm, 128), lambda i: (i, 0)),
                  pl.BlockSpec((bm, 128), lambda i: (i, 0)),
                  pl.BlockSpec((bm, 1), lambda i: (i, 0)),
                  pl.BlockSpec((1, 128), lambda i: (0, 0))],
        out_specs=pl.BlockSpec((bm, 128), lambda i: (i, 0)),
        out_shape=jax.ShapeDtypeStruct((n, 128), jnp.float32),
    )(q0, q1, h, d2, b)


def kernel(x, edge_index, edge_weight, W1, b1, W2, b2):
    n, d = x.shape
    e = edge_index.shape[1]

    n_pad = ((n + NW * L - 1) // (NW * L)) * (NW * L)
    rows = -(-e // 128)
    rows128 = ((rows + NW * 16 - 1) // (NW * 16)) * (NW * 16)

    row = edge_index[0].astype(jnp.int32)
    col = edge_index[1].astype(jnp.int32)
    ew = edge_weight.reshape(-1).astype(jnp.float32)
    e_pad = rows128 * 128
    row2d = jnp.pad(row, (0, e_pad - e)).reshape(rows128, 128)
    col2d = jnp.pad(col, (0, e_pad - e)).reshape(rows128, 128)
    ew2d = jnp.pad(ew, (0, e_pad - e)).reshape(rows128, 128)
    x_pad = jnp.pad(x, ((0, n_pad - n), (0, 0)))

    degp = _deg_kernel(n_pad, rows128)(col2d, ew2d)
    deg0 = degp[:n_pad].reshape(n_pad, 1)
    deg1 = degp[n_pad:].reshape(n_pad, 1)
    h1, dinv, d2 = _mm_deg(x_pad, W1, deg0, deg1, 512)
    norm2d = _norm_kernel(n_pad, rows128)(
        dinv.reshape(n_pad), row2d, col2d, ew2d)
    b1r = b1.reshape(1, d)
    b2r = b2.reshape(1, d)

    agg = _aggregate_kernel(n_pad, rows128)

    p = agg(h1, row2d, col2d, norm2d)
    h2 = _fused_relu_mm(p[:n_pad], p[n_pad:], h1, d2, b1r, W2, 512)
    q = agg(h2, row2d, col2d, norm2d)
    out = _epilogue(q[:n_pad], q[n_pad:], h2, d2, b2r, 512)
    return out[:n]


# speedup vs baseline: 2.8589x; 1.2277x over previous
import jax
import jax.numpy as jnp
from jax import lax
from jax.experimental import pallas as pl
from jax.experimental.pallas import tpu as pltpu
from jax.experimental.pallas import tpu_sc as plsc

NC = 2
NS = 16
L = 16
NW = NC * NS


def _mesh():
    return plsc.VectorSubcoreMesh(
        core_axis_name="c", subcore_axis_name="s", num_cores=NC,
        num_subcores=NS)


def _deg_kernel(n_pad, rows128):
    nrows_per_w = rows128 // NW
    npt = n_pad // NS

    def body(col_h, ew_h, deg_h, deg_s, colb, ewb, zb, sem):
        cid = lax.axis_index("c")
        sid = lax.axis_index("s")
        wid = cid * NS + sid

        def fill_z(i, _):
            zb[pl.ds(i * L, L)] = jnp.zeros((L,), jnp.float32)
            return 0
        lax.fori_loop(0, npt // L, fill_z, 0)
        pltpu.sync_copy(zb, deg_s.at[pl.ds(sid * npt, npt)])
        plsc.subcore_barrier()

        base = wid * nrows_per_w
        pltpu.sync_copy(col_h.at[pl.ds(base, nrows_per_w)], colb)
        pltpu.sync_copy(ew_h.at[pl.ds(base, nrows_per_w)], ewb)
        def deg_chunk(b, _):
            def deg_fire(j, _):
                pltpu.async_copy(ewb.at[b * 16 + j],
                                 deg_s.at[colb.at[b * 16 + j]], sem,
                                 add=True)
                return 0
            lax.fori_loop(0, 16, deg_fire, 0)
            def deg_drain(j, _):
                pltpu.make_async_copy(ewb.at[b * 16 + j],
                                      deg_s.at[colb.at[b * 16 + j]],
                                      sem).wait()
                return 0
            lax.fori_loop(0, 16, deg_drain, 0)
            return 0
        lax.fori_loop(0, nrows_per_w // 16, deg_chunk, 0)
        plsc.subcore_barrier()

        pltpu.sync_copy(deg_s.at[pl.ds(sid * npt, npt)],
                        deg_h.at[pl.ds(cid * n_pad + sid * npt, npt)])

    return pl.kernel(
        body,
        out_type=jax.ShapeDtypeStruct((NC * n_pad,), jnp.float32),
        mesh=_mesh(),
        scratch_types=[
            pltpu.VMEM_SHARED((n_pad,), jnp.float32),
            pltpu.VMEM((rows128 // NW, 128), jnp.int32),
            pltpu.VMEM((rows128 // NW, 128), jnp.float32),
            pltpu.VMEM((n_pad // NS,), jnp.float32),
            pltpu.SemaphoreType.DMA,
        ],
        compiler_params=pltpu.CompilerParams(needs_layout_passes=False),
        name="gcn_deg_sc",
    )


def _norm_kernel(n_pad, rows128):
    nrows_per_w = rows128 // NW

    def body(dinv_h, row_h, col_h, ew_h, norm_h, rowb, colb, ewb, normb,
             dinv_all):
        cid = lax.axis_index("c")
        sid = lax.axis_index("s")
        wid = cid * NS + sid

        base = wid * nrows_per_w
        pltpu.sync_copy(dinv_h, dinv_all)
        pltpu.sync_copy(row_h.at[pl.ds(base, nrows_per_w)], rowb)
        pltpu.sync_copy(col_h.at[pl.ds(base, nrows_per_w)], colb)
        pltpu.sync_copy(ew_h.at[pl.ds(base, nrows_per_w)], ewb)

        def norm_row(j, _):
            for g in range(8):
                rr = rowb[j, pl.ds(g * L, L)]
                cc = colb[j, pl.ds(g * L, L)]
                ev = ewb[j, pl.ds(g * L, L)]
                dr = plsc.load_gather(dinv_all, [rr])
                dc = plsc.load_gather(dinv_all, [cc])
                normb[j, pl.ds(g * L, L)] = dr * ev * dc
            return 0
        lax.fori_loop(0, nrows_per_w, norm_row, 0)
        pltpu.sync_copy(normb, norm_h.at[pl.ds(base, nrows_per_w)])

    return pl.kernel(
        body,
        out_type=jax.ShapeDtypeStruct((rows128, 128), jnp.float32),
        mesh=_mesh(),
        scratch_types=[
            pltpu.VMEM((rows128 // NW, 128), jnp.int32),
            pltpu.VMEM((rows128 // NW, 128), jnp.int32),
            pltpu.VMEM((rows128 // NW, 128), jnp.float32),
            pltpu.VMEM((rows128 // NW, 128), jnp.float32),
            pltpu.VMEM((n_pad,), jnp.float32),
        ],
        compiler_params=pltpu.CompilerParams(needs_layout_passes=False),
        name="gcn_norm_sc",
    )


def _aggregate_kernel(n_pad, rows64):
    tr = rows64 // NW
    npt = n_pad // NS
    assert tr % 16 == 0
    nsch = tr // 16

    def body(h_hbm, row_h, col_h, norm_h, out_h, acc_s, rowc, colc, normc,
             vb0, vb1, zbuf, g0, g1, s0, s1):
        bufs = [vb0, vb1]
        gsems = [g0, g1]
        ssems = [s0, s1]
        cid = lax.axis_index("c")
        sid = lax.axis_index("s")
        wid = cid * NS + sid
        base = wid * tr
        iotas = [lax.iota(jnp.int32, L) + g * L for g in range(4)]

        def zfill(r, _):
            for g in range(8):
                zbuf[r, pl.ds(g * L, L)] = jnp.zeros((L,), jnp.float32)
            return 0
        lax.fori_loop(0, 16, zfill, 0)
        def zcopy(k, _):
            pltpu.sync_copy(zbuf, acc_s.at[pl.ds(sid * npt + k * 16, 16)])
            return 0
        lax.fori_loop(0, npt // 16, zcopy, 0)
        plsc.subcore_barrier()

        def scale(k, j):
            nv = [normc[j, pl.ds(g * L, L)] for g in range(4)]
            def scale_f(f, _):
                for u in range(4):
                    fv = jnp.full((L,), f * 4 + u, jnp.int32)
                    for g in range(4):
                        vals = plsc.load_gather(bufs[k], [iotas[g], fv])
                        plsc.store_scatter(bufs[k], [iotas[g], fv],
                                           vals * nv[g])
                return 0
            lax.fori_loop(0, 32, scale_f, 0)

        def super_chunk(sc, _):
            pltpu.sync_copy(row_h.at[pl.ds(base + sc * 16, 16)], rowc)
            pltpu.sync_copy(col_h.at[pl.ds(base + sc * 16, 16)], colc)
            pltpu.sync_copy(norm_h.at[pl.ds(base + sc * 16, 16)], normc)
            pltpu.async_copy(h_hbm.at[rowc.at[0]], bufs[0], gsems[0])
            def pair(pi, _):
                for k in range(2):
                    j = pi * 2 + k
                    kp = (k + 1) % 2
                    @pl.when(j >= 1)
                    def _():
                        pltpu.make_async_copy(
                            bufs[kp], acc_s.at[colc.at[0]],
                            ssems[kp]).wait()
                    @pl.when(j + 1 < 16)
                    def _():
                        pltpu.async_copy(h_hbm.at[rowc.at[j + 1]],
                                         bufs[kp], gsems[kp])
                    pltpu.make_async_copy(h_hbm.at[rowc.at[0]], bufs[k],
                                          gsems[k]).wait()
                    scale(k, j)
                    pltpu.async_copy(bufs[k], acc_s.at[colc.at[j]],
                                     ssems[k], add=True)
                return 0
            lax.fori_loop(0, 8, pair, 0)
            pltpu.make_async_copy(bufs[1], acc_s.at[colc.at[0]],
                                  ssems[1]).wait()
            return 0
        lax.fori_loop(0, nsch, super_chunk, 0)
        plsc.subcore_barrier()

        pltpu.sync_copy(acc_s.at[pl.ds(sid * npt, npt)],
                        out_h.at[pl.ds(cid * n_pad + sid * npt, npt)])

    return pl.kernel(
        body,
        out_type=jax.ShapeDtypeStruct((NC * n_pad, 128), jnp.float32),
        mesh=_mesh(),
        scratch_types=[
            pltpu.VMEM_SHARED((n_pad, 128), jnp.float32),
            pltpu.VMEM((16, 64), jnp.int32),
            pltpu.VMEM((16, 64), jnp.int32),
            pltpu.VMEM((16, 64), jnp.float32),
            pltpu.VMEM((64, 128), jnp.float32),
            pltpu.VMEM((64, 128), jnp.float32),
            pltpu.VMEM((16, 128), jnp.float32),
            pltpu.SemaphoreType.DMA,
            pltpu.SemaphoreType.DMA,
            pltpu.SemaphoreType.DMA,
            pltpu.SemaphoreType.DMA,
        ],
        compiler_params=pltpu.CompilerParams(needs_layout_passes=False),
        name="gcn_aggregate_sc",
    )


def _mm_deg(x, w, deg0, deg1, bm):
    n = x.shape[0]
    def k(xb, wb, d0b, d1b, hb, dib, d2b):
        hb[...] = jnp.dot(xb[...], wb[...],
                          preferred_element_type=jnp.float32)
        dinv = lax.rsqrt(1.0 + d0b[...] + d1b[...])
        dib[...] = dinv
        d2b[...] = dinv * dinv
    return pl.pallas_call(
        k,
        grid=(n // bm,),
        in_specs=[pl.BlockSpec((bm, 128), lambda i: (i, 0)),
                  pl.BlockSpec((128, 128), lambda i: (0, 0)),
                  pl.BlockSpec((bm, 1), lambda i: (i, 0)),
                  pl.BlockSpec((bm, 1), lambda i: (i, 0))],
        out_specs=[pl.BlockSpec((bm, 128), lambda i: (i, 0)),
                   pl.BlockSpec((bm, 1), lambda i: (i, 0)),
                   pl.BlockSpec((bm, 1), lambda i: (i, 0))],
        out_shape=[jax.ShapeDtypeStruct((n, 128), jnp.float32),
                   jax.ShapeDtypeStruct((n, 1), jnp.float32),
                   jax.ShapeDtypeStruct((n, 1), jnp.float32)],
    )(x, w, deg0, deg1)


def _fused_relu_mm(p0, p1, h, d2, b, w, bm):
    n = h.shape[0]
    def k(p0b, p1b, hb, db, bb, wb, ob):
        g = jnp.maximum(
            p0b[...] + p1b[...] + db[...] * hb[...] + bb[...], 0.0)
        ob[...] = jnp.dot(g, wb[...], preferred_element_type=jnp.float32)
    return pl.pallas_call(
        k,
        grid=(n // bm,),
        in_specs=[pl.BlockSpec((bm, 128), lambda i: (i, 0)),
                  pl.BlockSpec((bm, 128), lambda i: (i, 0)),
                  pl.BlockSpec((bm, 128), lambda i: (i, 0)),
                  pl.BlockSpec((bm, 1), lambda i: (i, 0)),
                  pl.BlockSpec((1, 128), lambda i: (0, 0)),
                  pl.BlockSpec((128, 128), lambda i: (0, 0))],
        out_specs=pl.BlockSpec((bm, 128), lambda i: (i, 0)),
        out_shape=jax.ShapeDtypeStruct((n, 128), jnp.float32),
    )(p0, p1, h, d2, b, w)


def _epilogue(q0, q1, h, d2, b, bm):
    n = h.shape[0]
    def k(q0b, q1b, hb, db, bb, ob):
        ob[...] = q0b[...] + q1b[...] + db[...] * hb[...] + bb[...]
    return pl.pallas_call(
        k,
        grid=(n // bm,),
        in_specs=[pl.BlockSpec((bm, 128), lambda i: (i, 0)),
                  pl.BlockSpec((bm, 128), lambda i: (i, 0)),
                  pl.BlockSpec((bm, 128), lambda i: (i, 0)),
                  pl.BlockSpec((bm, 1), lambda i: (i, 0)),
                  pl.BlockSpec((1, 128), lambda i: (0, 0))],
        out_specs=pl.BlockSpec((bm, 128), lambda i: (i, 0)),
        out_shape=jax.ShapeDtypeStruct((n, 128), jnp.float32),
    )(q0, q1, h, d2, b)


def kernel(x, edge_index, edge_weight, W1, b1, W2, b2):
    n, d = x.shape
    e = edge_index.shape[1]

    n_pad = ((n + NW * L - 1) // (NW * L)) * (NW * L)
    rows = -(-e // 128)
    rows128 = ((rows + NW * 16 - 1) // (NW * 16)) * (NW * 16)

    row = edge_index[0].astype(jnp.int32)
    col = edge_index[1].astype(jnp.int32)
    ew = edge_weight.reshape(-1).astype(jnp.float32)
    e_pad = rows128 * 128
    row2d = jnp.pad(row, (0, e_pad - e)).reshape(rows128, 128)
    col2d = jnp.pad(col, (0, e_pad - e)).reshape(rows128, 128)
    ew2d = jnp.pad(ew, (0, e_pad - e)).reshape(rows128, 128)
    x_pad = jnp.pad(x, ((0, n_pad - n), (0, 0)))

    degp = _deg_kernel(n_pad, rows128)(col2d, ew2d)
    deg0 = degp[:n_pad].reshape(n_pad, 1)
    deg1 = degp[n_pad:].reshape(n_pad, 1)
    h1, dinv, d2 = _mm_deg(x_pad, W1, deg0, deg1, 512)
    norm2d = _norm_kernel(n_pad, rows128)(
        dinv.reshape(n_pad), row2d, col2d, ew2d)
    b1r = b1.reshape(1, d)
    b2r = b2.reshape(1, d)

    agg = _aggregate_kernel(n_pad, rows128 * 2)
    row64 = row2d.reshape(rows128 * 2, 64)
    col64 = col2d.reshape(rows128 * 2, 64)

    p = agg(h1, row64, col64, norm2d.reshape(rows128 * 2, 64))
    h2 = _fused_relu_mm(p[:n_pad], p[n_pad:], h1, d2, b1r, W2, 512)
    q = agg(h2, row64, col64, norm2d.reshape(rows128 * 2, 64))
    out = _epilogue(q[:n_pad], q[n_pad:], h2, d2, b2r, 512)
    return out[:n]
